# baseline (speedup 1.0000x reference)
import functools

import jax
import jax.numpy as jnp
from jax import lax
from jax.experimental import pallas as pl
from jax.experimental.pallas import tpu as pltpu
from jax.experimental.pallas import tpu_sc as plsc


def _filter_matmul(edge_basis_t, W_t, b2d, e0, ne):
    R, E = edge_basis_t.shape
    D = W_t.shape[1]
    BE = 2560

    def mm_kernel(a_ref, w_ref, b_ref, o_ref):
        o_ref[...] = lax.dot_general(
            a_ref[...], w_ref[...], (((0,), (0,)), ((), ())),
            preferred_element_type=jnp.float32) + b_ref[...]

    return pl.pallas_call(
        mm_kernel,
        grid=(ne // BE,),
        in_specs=[
            pl.BlockSpec((R, BE), lambda i: (0, i + e0 // BE)),
            pl.BlockSpec((R, D), lambda i: (0, 0)),
            pl.BlockSpec((1, D), lambda i: (0, 0)),
        ],
        out_specs=pl.BlockSpec((BE, D), lambda i: (i, 0)),
        out_shape=jax.ShapeDtypeStruct((ne, D), jnp.float32),
    )(edge_basis_t, W_t, b2d)


def _sc_gather_mul_scatter(x, eidx, filt, e0, ne):
    N, D = x.shape
    E = eidx.shape[0] // 2
    C = 48
    NW = 32
    EPT = ne // NW
    NT = EPT // C
    NT3 = NT // 3 * 3
    TAIL = EPT - NT * C
    NSUB = 16
    RPT_A = 624
    RPT_LAST = N - (NSUB - 1) * RPT_A
    NVEC = D // 16

    mesh = plsc.VectorSubcoreMesh(core_axis_name="c", subcore_axis_name="s")

    @functools.partial(
        pl.kernel,
        out_type=jax.ShapeDtypeStruct((2, N, D), jnp.float32),
        mesh=mesh,
        scratch_types=[
            pltpu.VMEM((EPT,), jnp.int32),
            pltpu.VMEM((C, D), jnp.float32),
            pltpu.VMEM((C, D), jnp.float32),
            pltpu.VMEM((C, D), jnp.float32),
            pltpu.VMEM((C, D), jnp.float32),
            pltpu.VMEM((C, D), jnp.float32),
            pltpu.VMEM((C, D), jnp.float32),
            pltpu.VMEM((C,), jnp.int32),
            pltpu.VMEM((C,), jnp.int32),
            pltpu.VMEM((C,), jnp.int32),
            pltpu.VMEM((max(TAIL, 8),), jnp.int32),
            pltpu.VMEM_SHARED((N, D), jnp.float32),
            pltpu.SemaphoreType.DMA,
            pltpu.SemaphoreType.DMA,
            pltpu.SemaphoreType.DMA,
            pltpu.SemaphoreType.DMA,
            pltpu.SemaphoreType.DMA,
            pltpu.SemaphoreType.DMA,
            pltpu.SemaphoreType.DMA,
            pltpu.SemaphoreType.DMA,
            pltpu.SemaphoreType.DMA,
            pltpu.SemaphoreType.DMA,
            pltpu.SemaphoreType.DMA,
            pltpu.SemaphoreType.DMA,
        ],
    )
    def k(x_hbm, eidx_hbm, filt_hbm, out_hbm,
          src_all, xg0, xg1, xg2, f0, f1, f2, d0, d1, d2, dt,
          acc_sh, gs0, gs1, gs2, fs0, fs1, fs2, ss0, ss1, ss2,
          ds0, ds1, ds2):
        xg = [xg0, xg1, xg2]
        fb = [f0, f1, f2]
        dc = [d0, d1, d2]
        gsem = [gs0, gs1, gs2]
        fsem = [fs0, fs1, fs2]
        ssem = [ss0, ss1, ss2]
        dsem = [ds0, ds1, ds2]

        c = lax.axis_index("c")
        s = lax.axis_index("s")
        w = s * 2 + c
        ebase = w * EPT

        zero = jnp.zeros((16,), jnp.float32)

        @plsc.parallel_loop(0, C)
        def _(r):
            for kk in range(NVEC):
                xg0[r, pl.ds(kk * 16, 16)] = zero

        @pl.when(s < NSUB - 1)
        def _():
            for p in range(RPT_A // C):
                pltpu.sync_copy(
                    xg0.at[pl.ds(0, C)],
                    acc_sh.at[pl.ds(s * RPT_A + p * C, C)])

        @pl.when(s == NSUB - 1)
        def _():
            lbase = (NSUB - 1) * RPT_A
            for p in range(RPT_LAST // C):
                pltpu.sync_copy(
                    xg0.at[pl.ds(0, C)],
                    acc_sh.at[pl.ds(lbase + p * C, C)])
            rem = RPT_LAST % C
            if rem:
                pltpu.sync_copy(
                    xg0.at[pl.ds(0, rem)],
                    acc_sh.at[pl.ds(lbase + (RPT_LAST // C) * C, rem)])

        plsc.subcore_barrier()

        pltpu.sync_copy(eidx_hbm.at[pl.ds(e0 + ebase, EPT)], src_all)

        def issue(t, b):
            pltpu.async_copy(
                x_hbm.at[src_all.at[pl.ds(t * C, C)]], xg[b], gsem[b])
            pltpu.async_copy(
                filt_hbm.at[pl.ds(ebase + t * C, C)], fb[b], fsem[b])
            pltpu.async_copy(
                eidx_hbm.at[pl.ds(E + e0 + ebase + t * C, C)], dc[b], dsem[b])

        def wait_gather(b):
            pltpu.make_async_copy(
                x_hbm.at[src_all.at[pl.ds(0, C)]], xg[b], gsem[b]).wait()

        def wait_filt(b):
            pltpu.make_async_copy(
                filt_hbm.at[pl.ds(0, C)], fb[b], fsem[b]).wait()

        def wait_didx(b):
            pltpu.make_async_copy(
                eidx_hbm.at[pl.ds(0, C)], dc[b], dsem[b]).wait()

        def wait_scat(b):
            pltpu.make_async_copy(
                x_hbm.at[pl.ds(0, C)], xg[b], ssem[b]).wait()

        def mul_chunk(xgb, fbb, rows):
            @plsc.parallel_loop(0, rows)
            def _(r):
                for j in range(NVEC):
                    sl = pl.ds(j * 16, 16)
                    xgb[r, sl] = xgb[r, sl] * fbb[r, sl]

        issue(0, 0)
        issue(1, 1)

        def outer(g, _):
            for b in range(3):
                t = 3 * g + b
                wait_gather(b)
                wait_filt(b)
                wait_didx(b)
                mul_chunk(xg[b], fb[b], C)
                pltpu.async_copy(xg[b], acc_sh.at[dc[b]], ssem[b], add=True)

                tn = t + 2
                bn = (b + 2) % 3

                @pl.when(tn < NT3)
                def _():
                    @pl.when(tn >= 3)
                    def _():
                        wait_scat(bn)
                    issue(tn, bn)

            return 0

        lax.fori_loop(0, NT3 // 3, outer, 0)
        for b in range(3):
            wait_scat(b)

        sizes = [C] * (NT - NT3) + ([TAIL] if TAIL else [])
        off = NT3 * C
        for sz in sizes:
            idxr = dc[0] if sz == C else dt
            pltpu.async_copy(
                x_hbm.at[src_all.at[pl.ds(off, sz)]],
                xg0.at[pl.ds(0, sz)], gs0).wait()
            pltpu.sync_copy(
                filt_hbm.at[pl.ds(ebase + off, sz)], f0.at[pl.ds(0, sz)])
            pltpu.sync_copy(eidx_hbm.at[pl.ds(E + e0 + ebase + off, sz)], idxr)
            mul_chunk(xg0, f0, sz)
            pltpu.sync_copy(xg0.at[pl.ds(0, sz)], acc_sh.at[idxr], add=True)
            off += sz

        plsc.subcore_barrier()

        @pl.when(s < NSUB - 1)
        def _():
            rbase = s * RPT_A
            pltpu.sync_copy(
                acc_sh.at[pl.ds(rbase, RPT_A)],
                out_hbm.at[c, pl.ds(rbase, RPT_A)])

        @pl.when(s == NSUB - 1)
        def _():
            rbase = (NSUB - 1) * RPT_A
            pltpu.sync_copy(
                acc_sh.at[pl.ds(rbase, RPT_LAST)],
                out_hbm.at[c, pl.ds(rbase, RPT_LAST)])

    return k(x, eidx, filt)


def _add_partials(pa, pb):
    _, N, D = pa.shape
    BN = 2000

    def add_k(pa_ref, pb_ref, o_ref):
        o_ref[...] = (pa_ref[0] + pa_ref[1]) + (pb_ref[0] + pb_ref[1])

    return pl.pallas_call(
        add_k,
        grid=(N // BN,),
        in_specs=[
            pl.BlockSpec((2, BN, D), lambda i: (0, i, 0)),
            pl.BlockSpec((2, BN, D), lambda i: (0, i, 0)),
        ],
        out_specs=pl.BlockSpec((BN, D), lambda i: (i, 0)),
        out_shape=jax.ShapeDtypeStruct((N, D), jnp.float32),
    )(pa, pb)


def kernel(x, edge_index, edge_basis, W, b):
    E = edge_index.shape[1]
    EA = 81920
    eidx = edge_index.reshape(-1)
    basis_t = edge_basis.T
    w_t = W.T
    b2d = b.reshape(1, -1)
    filt_a = _filter_matmul(basis_t, w_t, b2d, 0, EA)
    filt_b = _filter_matmul(basis_t, w_t, b2d, EA, E - EA)
    part_a = _sc_gather_mul_scatter(x, eidx, filt_a, 0, EA)
    part_b = _sc_gather_mul_scatter(x, eidx, filt_b, EA, E - EA)
    return _add_partials(part_a, part_b)


# speedup vs baseline: 1.1725x; 1.1725x over previous
import functools

import jax
import jax.numpy as jnp
from jax import lax
from jax.experimental import pallas as pl
from jax.experimental.pallas import tpu as pltpu
from jax.experimental.pallas import tpu_sc as plsc


def _filter_matmul(edge_basis_t, W_t, b2d, e0, ne):
    R, E = edge_basis_t.shape
    D = W_t.shape[1]
    BE = 2560

    def mm_kernel(a_ref, w_ref, b_ref, o_ref):
        o_ref[...] = lax.dot_general(
            a_ref[...], w_ref[...], (((0,), (0,)), ((), ())),
            preferred_element_type=jnp.float32) + b_ref[...]

    return pl.pallas_call(
        mm_kernel,
        grid=(ne // BE,),
        in_specs=[
            pl.BlockSpec((R, BE), lambda i: (0, i + e0 // BE)),
            pl.BlockSpec((R, D), lambda i: (0, 0)),
            pl.BlockSpec((1, D), lambda i: (0, 0)),
        ],
        out_specs=pl.BlockSpec((BE, D), lambda i: (i, 0)),
        out_shape=jax.ShapeDtypeStruct((ne, D), jnp.float32),
    )(edge_basis_t, W_t, b2d)


def _sc_gather_mul_scatter(x, eidx, filt, e0, ne):
    N, D = x.shape
    E = eidx.shape[0] // 2
    C = 48
    NW = 32
    EPT = ne // NW
    NT = EPT // C
    NT3 = NT // 3 * 3
    TAIL = EPT - NT * C
    NSUB = 16
    RPT_A = 624
    RPT_LAST = N - (NSUB - 1) * RPT_A
    NVEC = D // 16

    mesh = plsc.VectorSubcoreMesh(core_axis_name="c", subcore_axis_name="s")

    @functools.partial(
        pl.kernel,
        out_type=jax.ShapeDtypeStruct((2, N, D), jnp.float32),
        mesh=mesh,
        scratch_types=[
            pltpu.VMEM((EPT,), jnp.int32),
            pltpu.VMEM((C, D), jnp.float32),
            pltpu.VMEM((C, D), jnp.float32),
            pltpu.VMEM((C, D), jnp.float32),
            pltpu.VMEM((C, D), jnp.float32),
            pltpu.VMEM((C, D), jnp.float32),
            pltpu.VMEM((C, D), jnp.float32),
            pltpu.VMEM((C,), jnp.int32),
            pltpu.VMEM((C,), jnp.int32),
            pltpu.VMEM((C,), jnp.int32),
            pltpu.VMEM((max(TAIL, 8),), jnp.int32),
            pltpu.VMEM_SHARED((N, D), jnp.float32),
            pltpu.SemaphoreType.DMA,
            pltpu.SemaphoreType.DMA,
            pltpu.SemaphoreType.DMA,
            pltpu.SemaphoreType.DMA,
            pltpu.SemaphoreType.DMA,
            pltpu.SemaphoreType.DMA,
            pltpu.SemaphoreType.DMA,
            pltpu.SemaphoreType.DMA,
            pltpu.SemaphoreType.DMA,
            pltpu.SemaphoreType.DMA,
            pltpu.SemaphoreType.DMA,
            pltpu.SemaphoreType.DMA,
        ],
    )
    def k(x_hbm, eidx_hbm, filt_hbm, out_hbm,
          src_all, xg0, xg1, xg2, f0, f1, f2, d0, d1, d2, dt,
          acc_sh, gs0, gs1, gs2, fs0, fs1, fs2, ss0, ss1, ss2,
          ds0, ds1, ds2):
        xg = [xg0, xg1, xg2]
        fb = [f0, f1, f2]
        dc = [d0, d1, d2]
        gsem = [gs0, gs1, gs2]
        fsem = [fs0, fs1, fs2]
        ssem = [ss0, ss1, ss2]
        dsem = [ds0, ds1, ds2]

        c = lax.axis_index("c")
        s = lax.axis_index("s")
        w = s * 2 + c
        ebase = w * EPT

        zero = jnp.zeros((16,), jnp.float32)

        @plsc.parallel_loop(0, C)
        def _(r):
            for kk in range(NVEC):
                xg0[r, pl.ds(kk * 16, 16)] = zero

        @pl.when(s < NSUB - 1)
        def _():
            for p in range(RPT_A // C):
                pltpu.sync_copy(
                    xg0.at[pl.ds(0, C)],
                    acc_sh.at[pl.ds(s * RPT_A + p * C, C)])

        @pl.when(s == NSUB - 1)
        def _():
            lbase = (NSUB - 1) * RPT_A
            for p in range(RPT_LAST // C):
                pltpu.sync_copy(
                    xg0.at[pl.ds(0, C)],
                    acc_sh.at[pl.ds(lbase + p * C, C)])
            rem = RPT_LAST % C
            if rem:
                pltpu.sync_copy(
                    xg0.at[pl.ds(0, rem)],
                    acc_sh.at[pl.ds(lbase + (RPT_LAST // C) * C, rem)])

        plsc.subcore_barrier()

        pltpu.sync_copy(eidx_hbm.at[pl.ds(e0 + ebase, EPT)], src_all)

        def issue(t, b):
            pltpu.async_copy(
                x_hbm.at[src_all.at[pl.ds(t * C, C)]], xg[b], gsem[b])
            pltpu.async_copy(
                filt_hbm.at[pl.ds(ebase + t * C, C)], fb[b], fsem[b])
            pltpu.async_copy(
                eidx_hbm.at[pl.ds(E + e0 + ebase + t * C, C)], dc[b], dsem[b])

        def wait_gather(b):
            pltpu.make_async_copy(
                x_hbm.at[src_all.at[pl.ds(0, C)]], xg[b], gsem[b]).wait()

        def wait_filt(b):
            pltpu.make_async_copy(
                filt_hbm.at[pl.ds(0, C)], fb[b], fsem[b]).wait()

        def wait_didx(b):
            pltpu.make_async_copy(
                eidx_hbm.at[pl.ds(0, C)], dc[b], dsem[b]).wait()

        def wait_scat(b):
            pltpu.make_async_copy(
                x_hbm.at[pl.ds(0, C)], xg[b], ssem[b]).wait()

        def mul_chunk(xgb, fbb, rows):
            @plsc.parallel_loop(0, rows)
            def _(r):
                for j in range(NVEC):
                    sl = pl.ds(j * 16, 16)
                    xgb[r, sl] = xgb[r, sl] * fbb[r, sl]

        issue(0, 0)
        issue(1, 1)

        def outer(g, _):
            for b in range(3):
                t = 3 * g + b
                wait_gather(b)
                wait_filt(b)
                wait_didx(b)
                mul_chunk(xg[b], fb[b], C)
                pltpu.async_copy(xg[b], acc_sh.at[dc[b]], ssem[b], add=True)

                tn = t + 2
                bn = (b + 2) % 3

                @pl.when(tn < NT3)
                def _():
                    @pl.when(tn >= 3)
                    def _():
                        wait_scat(bn)
                    issue(tn, bn)

            return 0

        lax.fori_loop(0, NT3 // 3, outer, 0)
        for b in range(3):
            wait_scat(b)

        sizes = [C] * (NT - NT3) + ([TAIL] if TAIL else [])
        off = NT3 * C
        for sz in sizes:
            idxr = dc[0] if sz == C else dt
            pltpu.async_copy(
                x_hbm.at[src_all.at[pl.ds(off, sz)]],
                xg0.at[pl.ds(0, sz)], gs0).wait()
            pltpu.sync_copy(
                filt_hbm.at[pl.ds(ebase + off, sz)], f0.at[pl.ds(0, sz)])
            pltpu.sync_copy(eidx_hbm.at[pl.ds(E + e0 + ebase + off, sz)], idxr)
            mul_chunk(xg0, f0, sz)
            pltpu.sync_copy(xg0.at[pl.ds(0, sz)], acc_sh.at[idxr], add=True)
            off += sz

        plsc.subcore_barrier()

        @pl.when(s < NSUB - 1)
        def _():
            rbase = s * RPT_A
            pltpu.sync_copy(
                acc_sh.at[pl.ds(rbase, RPT_A)],
                out_hbm.at[c, pl.ds(rbase, RPT_A)])

        @pl.when(s == NSUB - 1)
        def _():
            rbase = (NSUB - 1) * RPT_A
            pltpu.sync_copy(
                acc_sh.at[pl.ds(rbase, RPT_LAST)],
                out_hbm.at[c, pl.ds(rbase, RPT_LAST)])

    return k(x, eidx, filt)


def _add_partials(parts):
    _, N, D = parts[0].shape
    BN = 2000

    def add_k(*refs):
        o_ref = refs[-1]
        acc = refs[0][0] + refs[0][1]
        for r in refs[1:-1]:
            acc = acc + (r[0] + r[1])
        o_ref[...] = acc

    return pl.pallas_call(
        add_k,
        grid=(N // BN,),
        in_specs=[
            pl.BlockSpec((2, BN, D), lambda i: (0, i, 0)) for _ in parts
        ],
        out_specs=pl.BlockSpec((BN, D), lambda i: (i, 0)),
        out_shape=jax.ShapeDtypeStruct((N, D), jnp.float32),
    )(*parts)


def kernel(x, edge_index, edge_basis, W, b):
    E = edge_index.shape[1]
    sizes = (38400, 92160, 189440)
    eidx = edge_index.reshape(-1)
    basis_t = edge_basis.T
    w_t = W.T
    b2d = b.reshape(1, -1)
    parts = []
    e0 = 0
    filt = None
    for ne in sizes:
        if filt is not None:
            w_t, _ = lax.optimization_barrier((w_t, filt))
        filt = _filter_matmul(basis_t, w_t, b2d, e0, ne)
        parts.append(_sc_gather_mul_scatter(x, eidx, filt, e0, ne))
        e0 += ne
    return _add_partials(parts)


# speedup vs baseline: 1.1818x; 1.0080x over previous
import functools

import jax
import jax.numpy as jnp
from jax import lax
from jax.experimental import pallas as pl
from jax.experimental.pallas import tpu as pltpu
from jax.experimental.pallas import tpu_sc as plsc


def _filter_matmul(edge_basis_t, W_t, b2d, e0, ne):
    R, E = edge_basis_t.shape
    D = W_t.shape[1]
    BE = 2560

    def mm_kernel(a_ref, w_ref, b_ref, o_ref):
        o_ref[...] = lax.dot_general(
            a_ref[...], w_ref[...], (((0,), (0,)), ((), ())),
            preferred_element_type=jnp.float32) + b_ref[...]

    return pl.pallas_call(
        mm_kernel,
        grid=(ne // BE,),
        in_specs=[
            pl.BlockSpec((R, BE), lambda i: (0, i + e0 // BE)),
            pl.BlockSpec((R, D), lambda i: (0, 0)),
            pl.BlockSpec((1, D), lambda i: (0, 0)),
        ],
        out_specs=pl.BlockSpec((BE, D), lambda i: (i, 0)),
        out_shape=jax.ShapeDtypeStruct((ne, D), jnp.float32),
    )(edge_basis_t, W_t, b2d)


def _sc_gather_mul_scatter(x, eidx, filt, e0, ne):
    N, D = x.shape
    E = eidx.shape[0] // 2
    C = 48
    NW = 32
    EPT = ne // NW
    NT = EPT // C
    NT3 = NT // 3 * 3
    TAIL = EPT - NT * C
    NSUB = 16
    RPT_A = 624
    RPT_LAST = N - (NSUB - 1) * RPT_A
    NVEC = D // 16

    mesh = plsc.VectorSubcoreMesh(core_axis_name="c", subcore_axis_name="s")

    @functools.partial(
        pl.kernel,
        out_type=jax.ShapeDtypeStruct((2, N, D), jnp.float32),
        mesh=mesh,
        scratch_types=[
            pltpu.VMEM((EPT,), jnp.int32),
            pltpu.VMEM((C, D), jnp.float32),
            pltpu.VMEM((C, D), jnp.float32),
            pltpu.VMEM((C, D), jnp.float32),
            pltpu.VMEM((C, D), jnp.float32),
            pltpu.VMEM((C, D), jnp.float32),
            pltpu.VMEM((C, D), jnp.float32),
            pltpu.VMEM((C,), jnp.int32),
            pltpu.VMEM((C,), jnp.int32),
            pltpu.VMEM((C,), jnp.int32),
            pltpu.VMEM((max(TAIL, 8),), jnp.int32),
            pltpu.VMEM_SHARED((N, D), jnp.float32),
            pltpu.SemaphoreType.DMA,
            pltpu.SemaphoreType.DMA,
            pltpu.SemaphoreType.DMA,
            pltpu.SemaphoreType.DMA,
            pltpu.SemaphoreType.DMA,
            pltpu.SemaphoreType.DMA,
            pltpu.SemaphoreType.DMA,
            pltpu.SemaphoreType.DMA,
            pltpu.SemaphoreType.DMA,
            pltpu.SemaphoreType.DMA,
            pltpu.SemaphoreType.DMA,
            pltpu.SemaphoreType.DMA,
            pltpu.SemaphoreType.DMA,
        ],
    )
    def k(x_hbm, eidx_hbm, filt_hbm, out_hbm,
          src_all, xg0, xg1, xg2, f0, f1, f2, d0, d1, d2, dt,
          acc_sh, zsem, gs0, gs1, gs2, fs0, fs1, fs2, ss0, ss1, ss2,
          ds0, ds1, ds2):
        xg = [xg0, xg1, xg2]
        fb = [f0, f1, f2]
        dc = [d0, d1, d2]
        gsem = [gs0, gs1, gs2]
        fsem = [fs0, fs1, fs2]
        ssem = [ss0, ss1, ss2]
        dsem = [ds0, ds1, ds2]

        c = lax.axis_index("c")
        s = lax.axis_index("s")
        w = s * 2 + c
        ebase = w * EPT

        def issue(t, b):
            pltpu.async_copy(
                x_hbm.at[src_all.at[pl.ds(t * C, C)]], xg[b], gsem[b])
            pltpu.async_copy(
                filt_hbm.at[pl.ds(ebase + t * C, C)], fb[b], fsem[b])
            pltpu.async_copy(
                eidx_hbm.at[pl.ds(E + e0 + ebase + t * C, C)], dc[b], dsem[b])

        pltpu.sync_copy(eidx_hbm.at[pl.ds(e0 + ebase, EPT)], src_all)
        issue(0, 0)
        issue(1, 1)

        zero = jnp.zeros((16,), jnp.float32)

        @plsc.parallel_loop(0, C)
        def _(r):
            for kk in range(NVEC):
                xg2[r, pl.ds(kk * 16, 16)] = zero

        @pl.when(s < NSUB - 1)
        def _():
            for p in range(RPT_A // C):
                pltpu.async_copy(
                    xg2, acc_sh.at[pl.ds(s * RPT_A + p * C, C)], zsem)
            for p in range(RPT_A // C):
                pltpu.make_async_copy(
                    xg2, acc_sh.at[pl.ds(0, C)], zsem).wait()

        @pl.when(s == NSUB - 1)
        def _():
            lbase = (NSUB - 1) * RPT_A
            nfull = RPT_LAST // C
            rem = RPT_LAST % C
            for p in range(nfull):
                pltpu.async_copy(
                    xg2, acc_sh.at[pl.ds(lbase + p * C, C)], zsem)
            if rem:
                pltpu.async_copy(
                    xg2.at[pl.ds(0, rem)],
                    acc_sh.at[pl.ds(lbase + nfull * C, rem)], zsem)
            for p in range(nfull):
                pltpu.make_async_copy(
                    xg2, acc_sh.at[pl.ds(0, C)], zsem).wait()
            if rem:
                pltpu.make_async_copy(
                    xg2.at[pl.ds(0, rem)],
                    acc_sh.at[pl.ds(0, rem)], zsem).wait()

        plsc.subcore_barrier()

        def wait_gather(b):
            pltpu.make_async_copy(
                x_hbm.at[src_all.at[pl.ds(0, C)]], xg[b], gsem[b]).wait()

        def wait_filt(b):
            pltpu.make_async_copy(
                filt_hbm.at[pl.ds(0, C)], fb[b], fsem[b]).wait()

        def wait_didx(b):
            pltpu.make_async_copy(
                eidx_hbm.at[pl.ds(0, C)], dc[b], dsem[b]).wait()

        def wait_scat(b):
            pltpu.make_async_copy(
                x_hbm.at[pl.ds(0, C)], xg[b], ssem[b]).wait()

        def mul_chunk(xgb, fbb, rows):
            @plsc.parallel_loop(0, rows)
            def _(r):
                for j in range(NVEC):
                    sl = pl.ds(j * 16, 16)
                    xgb[r, sl] = xgb[r, sl] * fbb[r, sl]

        def outer(g, _):
            for b in range(3):
                t = 3 * g + b
                wait_gather(b)
                wait_filt(b)
                wait_didx(b)
                mul_chunk(xg[b], fb[b], C)
                pltpu.async_copy(xg[b], acc_sh.at[dc[b]], ssem[b], add=True)

                tn = t + 2
                bn = (b + 2) % 3

                @pl.when(tn < NT3)
                def _():
                    @pl.when(tn >= 3)
                    def _():
                        wait_scat(bn)
                    issue(tn, bn)

            return 0

        lax.fori_loop(0, NT3 // 3, outer, 0)
        for b in range(3):
            wait_scat(b)

        sizes = [C] * (NT - NT3) + ([TAIL] if TAIL else [])
        off = NT3 * C
        for sz in sizes:
            idxr = dc[0] if sz == C else dt
            pltpu.async_copy(
                x_hbm.at[src_all.at[pl.ds(off, sz)]],
                xg0.at[pl.ds(0, sz)], gs0).wait()
            pltpu.sync_copy(
                filt_hbm.at[pl.ds(ebase + off, sz)], f0.at[pl.ds(0, sz)])
            pltpu.sync_copy(eidx_hbm.at[pl.ds(E + e0 + ebase + off, sz)], idxr)
            mul_chunk(xg0, f0, sz)
            pltpu.sync_copy(xg0.at[pl.ds(0, sz)], acc_sh.at[idxr], add=True)
            off += sz

        plsc.subcore_barrier()

        @pl.when(s < NSUB - 1)
        def _():
            rbase = s * RPT_A
            pltpu.sync_copy(
                acc_sh.at[pl.ds(rbase, RPT_A)],
                out_hbm.at[c, pl.ds(rbase, RPT_A)])

        @pl.when(s == NSUB - 1)
        def _():
            rbase = (NSUB - 1) * RPT_A
            pltpu.sync_copy(
                acc_sh.at[pl.ds(rbase, RPT_LAST)],
                out_hbm.at[c, pl.ds(rbase, RPT_LAST)])

    return k(x, eidx, filt)


def _add_partials(parts):
    _, N, D = parts[0].shape
    BN = 2000

    def add_k(*refs):
        o_ref = refs[-1]
        acc = refs[0][0] + refs[0][1]
        for r in refs[1:-1]:
            acc = acc + (r[0] + r[1])
        o_ref[...] = acc

    return pl.pallas_call(
        add_k,
        grid=(N // BN,),
        in_specs=[
            pl.BlockSpec((2, BN, D), lambda i: (0, i, 0)) for _ in parts
        ],
        out_specs=pl.BlockSpec((BN, D), lambda i: (i, 0)),
        out_shape=jax.ShapeDtypeStruct((N, D), jnp.float32),
    )(*parts)


def kernel(x, edge_index, edge_basis, W, b):
    E = edge_index.shape[1]
    sizes = (38400, 92160, 189440)
    eidx = edge_index.reshape(-1)
    basis_t = edge_basis.T
    w_t = W.T
    b2d = b.reshape(1, -1)
    parts = []
    e0 = 0
    filt = None
    for ne in sizes:
        if filt is not None:
            w_t, _ = lax.optimization_barrier((w_t, filt))
        filt = _filter_matmul(basis_t, w_t, b2d, e0, ne)
        parts.append(_sc_gather_mul_scatter(x, eidx, filt, e0, ne))
        e0 += ne
    return _add_partials(parts)
